# Initial kernel scaffold; baseline (speedup 1.0000x reference)
import jax
import jax.numpy as jnp
from jax.experimental import pallas as pl


def kernel(x, edge_index, batch, W1_0, b1_0, g1_0, be1_0, W2_0, b2_0, eps_0, go_0, bo_0, W1_1, b1_1, g1_1, be1_1, W2_1, b2_1, eps_1, go_1, bo_1):
    raise NotImplementedError("write your pallas kernel here")


# speedup vs baseline: 4.7831x; 4.7831x over previous
import functools

import jax
import jax.numpy as jnp
from jax import lax
from jax.experimental import pallas as pl
from jax.experimental.pallas import tpu as pltpu
from jax.experimental.pallas import tpu_sc as plsc

N = 10000
D = 128
E = 320000
NC = 2
NS = 16
NW = NC * NS
EPW = E // NW
K = 80
NCHUNK = EPW // K
RPT = 624
RTAIL = N - NS * RPT


def _sc_scatter_body(r_hbm, src_hbm, dst_hbm, zero_hbm, out_hbm,
                     src_v, dst_v, rows_v, acc_sh, sem):
    c = lax.axis_index("c")
    s = lax.axis_index("s")
    w = s * NC + c

    r0 = pl.multiple_of(s * RPT, 8)
    pltpu.sync_copy(zero_hbm.at[pl.ds(r0, RPT)], acc_sh.at[pl.ds(r0, RPT)])

    @pl.when(s == 0)
    def _zero_tail():
        pltpu.sync_copy(zero_hbm.at[pl.ds(NS * RPT, RTAIL)],
                        acc_sh.at[pl.ds(NS * RPT, RTAIL)])

    plsc.subcore_barrier()

    def chunk(i, carry):
        base = pl.multiple_of(w * EPW + i * K, 8)
        pltpu.sync_copy(src_hbm.at[pl.ds(base, K)], src_v)
        pltpu.sync_copy(dst_hbm.at[pl.ds(base, K)], dst_v)
        pltpu.async_copy(r_hbm.at[src_v], rows_v, sem).wait()
        pltpu.sync_copy(rows_v, acc_sh.at[dst_v], add=True)
        return carry

    lax.fori_loop(0, NCHUNK, chunk, 0)
    plsc.subcore_barrier()

    pltpu.sync_copy(acc_sh.at[pl.ds(r0, RPT)], out_hbm.at[c, pl.ds(r0, RPT)])

    @pl.when(s == 0)
    def _write_tail():
        pltpu.sync_copy(acc_sh.at[pl.ds(NS * RPT, RTAIL)],
                        out_hbm.at[c, pl.ds(NS * RPT, RTAIL)])


@functools.cache
def _make_sc_scatter():
    return pl.kernel(
        _sc_scatter_body,
        out_type=jax.ShapeDtypeStruct((NC, N, D), jnp.float32),
        mesh=plsc.VectorSubcoreMesh(core_axis_name="c", subcore_axis_name="s",
                                    num_cores=NC, num_subcores=NS),
        scratch_types=[
            pltpu.VMEM((K,), jnp.int32),
            pltpu.VMEM((K,), jnp.int32),
            pltpu.VMEM((K, D), jnp.float32),
            pltpu.VMEM_SHARED((N, D), jnp.float32),
            pltpu.SemaphoreType.DMA,
        ],
    )


def _sc_scatter(r, src, dst, zero):
    return _make_sc_scatter()(r, src, dst, zero)


def _relu_body(x_ref, o_ref):
    o_ref[...] = jnp.maximum(x_ref[...], 0.0)


def _relu(x):
    return pl.pallas_call(
        _relu_body,
        out_shape=jax.ShapeDtypeStruct((N, D), jnp.float32),
        grid=(10,),
        in_specs=[pl.BlockSpec((N // 10, D), lambda i: (i, 0))],
        out_specs=pl.BlockSpec((N // 10, D), lambda i: (i, 0)),
    )(x)


def _mlp_body(h_ref, pa_ref, pb_ref, eps_ref, W1_ref, b1_ref, g1_ref,
              be1_ref, W2_ref, b2_ref, go_ref, bo_ref, o_ref, *, final_relu):
    z = (1.0 + eps_ref[0]) * h_ref[...] + (pa_ref[...] + pb_ref[...])
    z1 = jnp.dot(z, W1_ref[...], preferred_element_type=jnp.float32) + b1_ref[...]
    m1 = jnp.mean(z1, axis=0, keepdims=True)
    v1 = jnp.mean((z1 - m1) ** 2, axis=0, keepdims=True)
    z1 = (z1 - m1) * lax.rsqrt(v1 + 1e-5) * g1_ref[...] + be1_ref[...]
    z1 = jnp.maximum(z1, 0.0)
    z2 = jnp.dot(z1, W2_ref[...], preferred_element_type=jnp.float32) + b2_ref[...]
    m2 = jnp.mean(z2, axis=0, keepdims=True)
    v2 = jnp.mean((z2 - m2) ** 2, axis=0, keepdims=True)
    z2 = (z2 - m2) * lax.rsqrt(v2 + 1e-5) * go_ref[...] + bo_ref[...]
    if final_relu:
        z2 = jnp.maximum(z2, 0.0)
    o_ref[...] = z2


def _mlp(h, pa, pb, eps, W1, b1, g1, be1, W2, b2, go, bo, final_relu):
    vec = lambda v: v.reshape(1, D)
    return pl.pallas_call(
        functools.partial(_mlp_body, final_relu=final_relu),
        out_shape=jax.ShapeDtypeStruct((N, D), jnp.float32),
        in_specs=[pl.BlockSpec(memory_space=pltpu.VMEM)] * 3
                 + [pl.BlockSpec(memory_space=pltpu.SMEM)]
                 + [pl.BlockSpec(memory_space=pltpu.VMEM)] * 8,
        out_specs=pl.BlockSpec(memory_space=pltpu.VMEM),
    )(h, pa, pb, eps.reshape(1), W1, vec(b1), vec(g1), vec(be1),
      W2, vec(b2), vec(go), vec(bo))


def kernel(x, edge_index, batch,
           W1_0, b1_0, g1_0, be1_0, W2_0, b2_0, eps_0, go_0, bo_0,
           W1_1, b1_1, g1_1, be1_1, W2_1, b2_1, eps_1, go_1, bo_1):
    src = edge_index[0]
    dst = edge_index[1]
    zero = jnp.zeros((N, D), jnp.float32)

    r0 = _relu(x)
    parts0 = _sc_scatter(r0, src, dst, zero)
    h1 = _mlp(x, parts0[0], parts0[1], eps_0,
              W1_0, b1_0, g1_0, be1_0, W2_0, b2_0, go_0, bo_0,
              final_relu=True)
    parts1 = _sc_scatter(h1, src, dst, zero)
    h2 = _mlp(h1, parts1[0], parts1[1], eps_1,
              W1_1, b1_1, g1_1, be1_1, W2_1, b2_1, go_1, bo_1,
              final_relu=False)
    return h2


# speedup vs baseline: 9.0011x; 1.8818x over previous
import functools

import jax
import jax.numpy as jnp
from jax import lax
from jax.experimental import pallas as pl
from jax.experimental.pallas import tpu as pltpu
from jax.experimental.pallas import tpu_sc as plsc

N = 10000
D = 128
E = 320000
NC = 2
NS = 16
NW = NC * NS
EPW = E // NW
K = 80
NCHUNK = EPW // K
RPT = 624
RTAIL = N - NS * RPT


def _sc_scatter_body(r_hbm, src_hbm, dst_hbm, zero_hbm, out_hbm,
                     src0, dst0, src1, dst1, rows0, rows1, acc_sh,
                     isem0, isem1, gsem0, gsem1):
    c = lax.axis_index("c")
    s = lax.axis_index("s")
    w = s * NC + c
    ebase = w * EPW

    def idx_start(i, src_v, dst_v, isem):
        base = pl.multiple_of(ebase + i * K, 8)
        pltpu.async_copy(src_hbm.at[pl.ds(base, K)], src_v, isem)
        pltpu.async_copy(dst_hbm.at[pl.ds(base, K)], dst_v, isem)

    def idx_wait(src_v, dst_v, isem):
        pltpu.make_async_copy(src_hbm.at[pl.ds(0, K)], src_v, isem).wait()
        pltpu.make_async_copy(dst_hbm.at[pl.ds(0, K)], dst_v, isem).wait()

    idx_start(0, src0, dst0, isem0)
    idx_start(1, src1, dst1, isem1)

    r0 = pl.multiple_of(s * RPT, 8)
    pltpu.sync_copy(zero_hbm.at[pl.ds(r0, RPT)], acc_sh.at[pl.ds(r0, RPT)])

    @pl.when(s == 0)
    def _zero_tail():
        pltpu.sync_copy(zero_hbm.at[pl.ds(NS * RPT, RTAIL)],
                        acc_sh.at[pl.ds(NS * RPT, RTAIL)])

    idx_wait(src0, dst0, isem0)
    pltpu.async_copy(r_hbm.at[src0], rows0, gsem0)
    plsc.subcore_barrier()

    def pair(j, carry):
        i0 = 2 * j
        idx_wait(src1, dst1, isem1)
        pltpu.async_copy(r_hbm.at[src1], rows1, gsem1)
        pltpu.make_async_copy(r_hbm.at[src0], rows0, gsem0).wait()
        pltpu.sync_copy(rows0, acc_sh.at[dst0], add=True)
        idx_start(jnp.minimum(i0 + 2, NCHUNK - 1), src0, dst0, isem0)
        idx_wait(src0, dst0, isem0)
        pltpu.async_copy(r_hbm.at[src0], rows0, gsem0)
        pltpu.make_async_copy(r_hbm.at[src1], rows1, gsem1).wait()
        pltpu.sync_copy(rows1, acc_sh.at[dst1], add=True)
        idx_start(jnp.minimum(i0 + 3, NCHUNK - 1), src1, dst1, isem1)
        return carry

    lax.fori_loop(0, NCHUNK // 2, pair, 0)
    pltpu.make_async_copy(r_hbm.at[src0], rows0, gsem0).wait()
    pltpu.sync_copy(rows0, acc_sh.at[dst0], add=True)
    idx_wait(src1, dst1, isem1)
    plsc.subcore_barrier()

    pltpu.sync_copy(acc_sh.at[pl.ds(r0, RPT)], out_hbm.at[c, pl.ds(r0, RPT)])

    @pl.when(s == 0)
    def _write_tail():
        pltpu.sync_copy(acc_sh.at[pl.ds(NS * RPT, RTAIL)],
                        out_hbm.at[c, pl.ds(NS * RPT, RTAIL)])


@functools.cache
def _make_sc_scatter():
    return pl.kernel(
        _sc_scatter_body,
        out_type=jax.ShapeDtypeStruct((NC, N, D), jnp.float32),
        mesh=plsc.VectorSubcoreMesh(core_axis_name="c", subcore_axis_name="s",
                                    num_cores=NC, num_subcores=NS),
        scratch_types=[
            pltpu.VMEM((K,), jnp.int32),
            pltpu.VMEM((K,), jnp.int32),
            pltpu.VMEM((K,), jnp.int32),
            pltpu.VMEM((K,), jnp.int32),
            pltpu.VMEM((K, D), jnp.float32),
            pltpu.VMEM((K, D), jnp.float32),
            pltpu.VMEM_SHARED((N, D), jnp.float32),
            pltpu.SemaphoreType.DMA,
            pltpu.SemaphoreType.DMA,
            pltpu.SemaphoreType.DMA,
            pltpu.SemaphoreType.DMA,
        ],
    )


def _sc_scatter(r, src, dst, zero):
    return _make_sc_scatter()(r, src, dst, zero)


def _relu_body(x_ref, o_ref):
    o_ref[...] = jnp.maximum(x_ref[...], 0.0)


def _relu(x):
    return pl.pallas_call(
        _relu_body,
        out_shape=jax.ShapeDtypeStruct((N, D), jnp.float32),
        grid=(10,),
        in_specs=[pl.BlockSpec((N // 10, D), lambda i: (i, 0))],
        out_specs=pl.BlockSpec((N // 10, D), lambda i: (i, 0)),
    )(x)


def _mlp_body(h_ref, pa_ref, pb_ref, eps_ref, W1_ref, b1_ref, g1_ref,
              be1_ref, W2_ref, b2_ref, go_ref, bo_ref, o_ref, *, final_relu):
    z = (1.0 + eps_ref[0]) * h_ref[...] + (pa_ref[...] + pb_ref[...])
    z1 = jnp.dot(z, W1_ref[...], preferred_element_type=jnp.float32) + b1_ref[...]
    m1 = jnp.mean(z1, axis=0, keepdims=True)
    v1 = jnp.mean((z1 - m1) ** 2, axis=0, keepdims=True)
    z1 = (z1 - m1) * lax.rsqrt(v1 + 1e-5) * g1_ref[...] + be1_ref[...]
    z1 = jnp.maximum(z1, 0.0)
    z2 = jnp.dot(z1, W2_ref[...], preferred_element_type=jnp.float32) + b2_ref[...]
    m2 = jnp.mean(z2, axis=0, keepdims=True)
    v2 = jnp.mean((z2 - m2) ** 2, axis=0, keepdims=True)
    z2 = (z2 - m2) * lax.rsqrt(v2 + 1e-5) * go_ref[...] + bo_ref[...]
    if final_relu:
        z2 = jnp.maximum(z2, 0.0)
    o_ref[...] = z2


def _mlp(h, pa, pb, eps, W1, b1, g1, be1, W2, b2, go, bo, final_relu):
    vec = lambda v: v.reshape(1, D)
    return pl.pallas_call(
        functools.partial(_mlp_body, final_relu=final_relu),
        out_shape=jax.ShapeDtypeStruct((N, D), jnp.float32),
        in_specs=[pl.BlockSpec(memory_space=pltpu.VMEM)] * 3
                 + [pl.BlockSpec(memory_space=pltpu.SMEM)]
                 + [pl.BlockSpec(memory_space=pltpu.VMEM)] * 8,
        out_specs=pl.BlockSpec(memory_space=pltpu.VMEM),
    )(h, pa, pb, eps.reshape(1), W1, vec(b1), vec(g1), vec(be1),
      W2, vec(b2), vec(go), vec(bo))


def kernel(x, edge_index, batch,
           W1_0, b1_0, g1_0, be1_0, W2_0, b2_0, eps_0, go_0, bo_0,
           W1_1, b1_1, g1_1, be1_1, W2_1, b2_1, eps_1, go_1, bo_1):
    src = edge_index[0]
    dst = edge_index[1]
    zero = jnp.zeros((N, D), jnp.float32)

    r0 = _relu(x)
    parts0 = _sc_scatter(r0, src, dst, zero)
    h1 = _mlp(x, parts0[0], parts0[1], eps_0,
              W1_0, b1_0, g1_0, be1_0, W2_0, b2_0, go_0, bo_0,
              final_relu=True)
    parts1 = _sc_scatter(h1, src, dst, zero)
    h2 = _mlp(h1, parts1[0], parts1[1], eps_1,
              W1_1, b1_1, g1_1, be1_1, W2_1, b2_1, go_1, bo_1,
              final_relu=False)
    return h2


# speedup vs baseline: 10.7929x; 1.1991x over previous
import functools

import jax
import jax.numpy as jnp
from jax import lax
from jax.experimental import pallas as pl
from jax.experimental.pallas import tpu as pltpu
from jax.experimental.pallas import tpu_sc as plsc

N = 10000
D = 128
E = 320000
NC = 2
NS = 16
NW = NC * NS
EPW = E // NW
K = 80
NCHUNK = EPW // K
RPT = 624
RTAIL = N - NS * RPT


def _sc_scatter_body(r_hbm, pk_hbm, zero_hbm, out_hbm,
                     q0, q1, q2, q3, rows0, rows1, acc_sh,
                     qs0, qs1, qs2, qs3, gsem0, gsem1):
    c = lax.axis_index("c")
    s = lax.axis_index("s")
    w = s * NC + c
    Q = (q0, q1, q2, q3)
    QS = (qs0, qs1, qs2, qs3)
    R = (rows0, rows1)
    GS = (gsem0, gsem1)

    def q_start(i, q, qsem):
        pltpu.async_copy(pk_hbm.at[w, i], q, qsem)

    def q_wait(q, qsem):
        pltpu.make_async_copy(pk_hbm.at[0, 0], q, qsem).wait()

    q_start(0, q0, qs0)
    q_start(1, q1, qs1)
    q_start(2, q2, qs2)

    r0 = pl.multiple_of(s * RPT, 8)
    pltpu.sync_copy(zero_hbm.at[pl.ds(r0, RPT)], acc_sh.at[pl.ds(r0, RPT)])

    @pl.when(s == 0)
    def _zero_tail():
        pltpu.sync_copy(zero_hbm.at[pl.ds(NS * RPT, RTAIL)],
                        acc_sh.at[pl.ds(NS * RPT, RTAIL)])

    q_wait(q0, qs0)
    pltpu.async_copy(r_hbm.at[q0.at[0]], rows0, gsem0)
    plsc.subcore_barrier()

    def quad(j, carry):
        i0 = 4 * j
        for b in range(4):
            i = i0 + b
            qn, qsn = Q[(b + 1) % 4], QS[(b + 1) % 4]
            q_wait(qn, qsn)
            pltpu.async_copy(r_hbm.at[qn.at[0]], R[(b + 1) % 2], GS[(b + 1) % 2])
            pltpu.make_async_copy(r_hbm.at[Q[b].at[0]], R[b % 2], GS[b % 2]).wait()
            pltpu.sync_copy(R[b % 2], acc_sh.at[Q[b].at[1]], add=True)
            q_start(jnp.minimum(i + 3, NCHUNK - 1), Q[(b + 3) % 4], QS[(b + 3) % 4])
        return carry

    lax.fori_loop(0, (NCHUNK - 1) // 4, quad, 0)
    pltpu.make_async_copy(r_hbm.at[q0.at[0]], rows0, gsem0).wait()
    pltpu.sync_copy(rows0, acc_sh.at[q0.at[1]], add=True)
    q_wait(q1, qs1)
    q_wait(q2, qs2)
    plsc.subcore_barrier()

    pltpu.sync_copy(acc_sh.at[pl.ds(r0, RPT)], out_hbm.at[c, pl.ds(r0, RPT)])

    @pl.when(s == 0)
    def _write_tail():
        pltpu.sync_copy(acc_sh.at[pl.ds(NS * RPT, RTAIL)],
                        out_hbm.at[c, pl.ds(NS * RPT, RTAIL)])


@functools.cache
def _make_sc_scatter():
    return pl.kernel(
        _sc_scatter_body,
        out_type=jax.ShapeDtypeStruct((NC, N, D), jnp.float32),
        mesh=plsc.VectorSubcoreMesh(core_axis_name="c", subcore_axis_name="s",
                                    num_cores=NC, num_subcores=NS),
        scratch_types=[
            pltpu.VMEM((2, K), jnp.int32),
            pltpu.VMEM((2, K), jnp.int32),
            pltpu.VMEM((2, K), jnp.int32),
            pltpu.VMEM((2, K), jnp.int32),
            pltpu.VMEM((K, D), jnp.float32),
            pltpu.VMEM((K, D), jnp.float32),
            pltpu.VMEM_SHARED((N, D), jnp.float32),
            pltpu.SemaphoreType.DMA,
            pltpu.SemaphoreType.DMA,
            pltpu.SemaphoreType.DMA,
            pltpu.SemaphoreType.DMA,
            pltpu.SemaphoreType.DMA,
            pltpu.SemaphoreType.DMA,
        ],
    )


def _sc_scatter(r, packed, zero):
    return _make_sc_scatter()(r, packed, zero)


def _relu_body(x_ref, o_ref):
    o_ref[...] = jnp.maximum(x_ref[...], 0.0)


def _relu(x):
    return pl.pallas_call(
        _relu_body,
        out_shape=jax.ShapeDtypeStruct((N, D), jnp.float32),
        grid=(10,),
        in_specs=[pl.BlockSpec((N // 10, D), lambda i: (i, 0))],
        out_specs=pl.BlockSpec((N // 10, D), lambda i: (i, 0)),
    )(x)


def _mlp_body(h_ref, pa_ref, pb_ref, eps_ref, W1_ref, b1_ref, g1_ref,
              be1_ref, W2_ref, b2_ref, go_ref, bo_ref, o_ref, *, final_relu):
    z = (1.0 + eps_ref[0]) * h_ref[...] + (pa_ref[...] + pb_ref[...])
    z1 = jnp.dot(z, W1_ref[...], preferred_element_type=jnp.float32) + b1_ref[...]
    m1 = jnp.mean(z1, axis=0, keepdims=True)
    v1 = jnp.mean((z1 - m1) ** 2, axis=0, keepdims=True)
    z1 = (z1 - m1) * lax.rsqrt(v1 + 1e-5) * g1_ref[...] + be1_ref[...]
    z1 = jnp.maximum(z1, 0.0)
    z2 = jnp.dot(z1, W2_ref[...], preferred_element_type=jnp.float32) + b2_ref[...]
    m2 = jnp.mean(z2, axis=0, keepdims=True)
    v2 = jnp.mean((z2 - m2) ** 2, axis=0, keepdims=True)
    z2 = (z2 - m2) * lax.rsqrt(v2 + 1e-5) * go_ref[...] + bo_ref[...]
    if final_relu:
        z2 = jnp.maximum(z2, 0.0)
    o_ref[...] = z2


def _mlp(h, pa, pb, eps, W1, b1, g1, be1, W2, b2, go, bo, final_relu):
    vec = lambda v: v.reshape(1, D)
    return pl.pallas_call(
        functools.partial(_mlp_body, final_relu=final_relu),
        out_shape=jax.ShapeDtypeStruct((N, D), jnp.float32),
        in_specs=[pl.BlockSpec(memory_space=pltpu.VMEM)] * 3
                 + [pl.BlockSpec(memory_space=pltpu.SMEM)]
                 + [pl.BlockSpec(memory_space=pltpu.VMEM)] * 8,
        out_specs=pl.BlockSpec(memory_space=pltpu.VMEM),
    )(h, pa, pb, eps.reshape(1), W1, vec(b1), vec(g1), vec(be1),
      W2, vec(b2), vec(go), vec(bo))


def kernel(x, edge_index, batch,
           W1_0, b1_0, g1_0, be1_0, W2_0, b2_0, eps_0, go_0, bo_0,
           W1_1, b1_1, g1_1, be1_1, W2_1, b2_1, eps_1, go_1, bo_1):
    packed = edge_index.reshape(2, NW, NCHUNK, K).transpose(1, 2, 0, 3)
    zero = jnp.zeros((N, D), jnp.float32)

    r0 = _relu(x)
    parts0 = _sc_scatter(r0, packed, zero)
    h1 = _mlp(x, parts0[0], parts0[1], eps_0,
              W1_0, b1_0, g1_0, be1_0, W2_0, b2_0, go_0, bo_0,
              final_relu=True)
    parts1 = _sc_scatter(h1, packed, zero)
    h2 = _mlp(h1, parts1[0], parts1[1], eps_1,
              W1_1, b1_1, g1_1, be1_1, W2_1, b2_1, go_1, bo_1,
              final_relu=False)
    return h2


# speedup vs baseline: 11.3943x; 1.0557x over previous
import functools

import jax
import jax.numpy as jnp
from jax import lax
from jax.experimental import pallas as pl
from jax.experimental.pallas import tpu as pltpu
from jax.experimental.pallas import tpu_sc as plsc

N = 10000
D = 128
E = 320000
NC = 2
NS = 16
NW = NC * NS
EPW = E // NW
K = 80
NCHUNK = EPW // K
RPT = 624
RTAIL = N - NS * RPT


def _sc_scatter_body(r_hbm, src_hbm, dst_hbm, out_hbm,
                     sq0, sq1, sq2, sq3, dq0, dq1, dq2, dq3,
                     rows0, rows1, acc_sh,
                     qs0, qs1, qs2, qs3, gsem0, gsem1):
    c = lax.axis_index("c")
    s = lax.axis_index("s")
    w = s * NC + c
    ebase = w * EPW
    SQ = (sq0, sq1, sq2, sq3)
    DQ = (dq0, dq1, dq2, dq3)
    QS = (qs0, qs1, qs2, qs3)
    R = (rows0, rows1)
    GS = (gsem0, gsem1)

    def q_start(i, k):
        base = pl.multiple_of(ebase + i * K, 8)
        pltpu.async_copy(src_hbm.at[pl.ds(base, K)], SQ[k], QS[k])
        pltpu.async_copy(dst_hbm.at[pl.ds(base, K)], DQ[k], QS[k])

    def q_wait(k):
        pltpu.make_async_copy(src_hbm.at[pl.ds(0, K)], SQ[k], QS[k]).wait()
        pltpu.make_async_copy(dst_hbm.at[pl.ds(0, K)], DQ[k], QS[k]).wait()

    q_start(0, 0)
    q_start(1, 1)
    q_start(2, 2)

    zv = jnp.zeros((16,), jnp.float32)

    def zrow(i, carry):
        for b in range(D // 16):
            rows1[i, pl.ds(b * 16, 16)] = zv
        return carry

    lax.fori_loop(0, K, zrow, 0)
    r0 = pl.multiple_of(s * RPT, 8)
    for t in range(RPT // K):
        pltpu.sync_copy(rows1, acc_sh.at[pl.ds(r0 + t * K, K)])
    pltpu.sync_copy(rows1.at[pl.ds(0, RPT - (RPT // K) * K)],
                    acc_sh.at[pl.ds(r0 + (RPT // K) * K, RPT - (RPT // K) * K)])

    @pl.when(s == 0)
    def _zero_tail():
        pltpu.sync_copy(rows1.at[pl.ds(0, RTAIL)],
                        acc_sh.at[pl.ds(NS * RPT, RTAIL)])

    q_wait(0)
    pltpu.async_copy(r_hbm.at[sq0], rows0, gsem0)
    plsc.subcore_barrier()

    def quad(j, carry):
        i0 = 4 * j
        for b in range(4):
            i = i0 + b
            q_wait((b + 1) % 4)
            pltpu.async_copy(r_hbm.at[SQ[(b + 1) % 4]], R[(b + 1) % 2],
                             GS[(b + 1) % 2])
            pltpu.make_async_copy(r_hbm.at[SQ[b]], R[b % 2], GS[b % 2]).wait()
            pltpu.sync_copy(R[b % 2], acc_sh.at[DQ[b]], add=True)
            q_start(jnp.minimum(i + 3, NCHUNK - 1), (b + 3) % 4)
        return carry

    lax.fori_loop(0, (NCHUNK - 1) // 4, quad, 0)
    pltpu.make_async_copy(r_hbm.at[sq0], rows0, gsem0).wait()
    pltpu.sync_copy(rows0, acc_sh.at[dq0], add=True)
    q_wait(1)
    q_wait(2)
    plsc.subcore_barrier()

    pltpu.sync_copy(acc_sh.at[pl.ds(r0, RPT)], out_hbm.at[c, pl.ds(r0, RPT)])

    @pl.when(s == 0)
    def _write_tail():
        pltpu.sync_copy(acc_sh.at[pl.ds(NS * RPT, RTAIL)],
                        out_hbm.at[c, pl.ds(NS * RPT, RTAIL)])


@functools.cache
def _make_sc_scatter():
    return pl.kernel(
        _sc_scatter_body,
        out_type=jax.ShapeDtypeStruct((NC, N, D), jnp.float32),
        mesh=plsc.VectorSubcoreMesh(core_axis_name="c", subcore_axis_name="s",
                                    num_cores=NC, num_subcores=NS),
        scratch_types=[
            pltpu.VMEM((K,), jnp.int32),
            pltpu.VMEM((K,), jnp.int32),
            pltpu.VMEM((K,), jnp.int32),
            pltpu.VMEM((K,), jnp.int32),
            pltpu.VMEM((K,), jnp.int32),
            pltpu.VMEM((K,), jnp.int32),
            pltpu.VMEM((K,), jnp.int32),
            pltpu.VMEM((K,), jnp.int32),
            pltpu.VMEM((K, D), jnp.float32),
            pltpu.VMEM((K, D), jnp.float32),
            pltpu.VMEM_SHARED((N, D), jnp.float32),
            pltpu.SemaphoreType.DMA,
            pltpu.SemaphoreType.DMA,
            pltpu.SemaphoreType.DMA,
            pltpu.SemaphoreType.DMA,
            pltpu.SemaphoreType.DMA,
            pltpu.SemaphoreType.DMA,
        ],
    )


def _sc_scatter(r, src, dst):
    return _make_sc_scatter()(r, src, dst)


def _relu_body(x_ref, o_ref):
    o_ref[...] = jnp.maximum(x_ref[...], 0.0)


def _relu(x):
    return pl.pallas_call(
        _relu_body,
        out_shape=jax.ShapeDtypeStruct((N, D), jnp.float32),
        grid=(10,),
        in_specs=[pl.BlockSpec((N // 10, D), lambda i: (i, 0))],
        out_specs=pl.BlockSpec((N // 10, D), lambda i: (i, 0)),
    )(x)


def _mlp_body(h_ref, parts_ref, eps_ref, W1_ref, b1_ref, g1_ref,
              be1_ref, W2_ref, b2_ref, go_ref, bo_ref, o_ref, *, final_relu):
    z = (1.0 + eps_ref[0]) * h_ref[...] + (parts_ref[0] + parts_ref[1])
    z1 = jnp.dot(z, W1_ref[...], preferred_element_type=jnp.float32) + b1_ref[...]
    m1 = jnp.mean(z1, axis=0, keepdims=True)
    v1 = jnp.mean((z1 - m1) ** 2, axis=0, keepdims=True)
    z1 = (z1 - m1) * lax.rsqrt(v1 + 1e-5) * g1_ref[...] + be1_ref[...]
    z1 = jnp.maximum(z1, 0.0)
    z2 = jnp.dot(z1, W2_ref[...], preferred_element_type=jnp.float32) + b2_ref[...]
    m2 = jnp.mean(z2, axis=0, keepdims=True)
    v2 = jnp.mean((z2 - m2) ** 2, axis=0, keepdims=True)
    z2 = (z2 - m2) * lax.rsqrt(v2 + 1e-5) * go_ref[...] + bo_ref[...]
    if final_relu:
        z2 = jnp.maximum(z2, 0.0)
    o_ref[...] = z2


def _mlp(h, parts, eps, W1, b1, g1, be1, W2, b2, go, bo, final_relu):
    vec = lambda v: v.reshape(1, D)
    return pl.pallas_call(
        functools.partial(_mlp_body, final_relu=final_relu),
        out_shape=jax.ShapeDtypeStruct((N, D), jnp.float32),
        in_specs=[pl.BlockSpec(memory_space=pltpu.VMEM)] * 2
                 + [pl.BlockSpec(memory_space=pltpu.SMEM)]
                 + [pl.BlockSpec(memory_space=pltpu.VMEM)] * 8,
        out_specs=pl.BlockSpec(memory_space=pltpu.VMEM),
    )(h, parts, eps.reshape(1), W1, vec(b1), vec(g1), vec(be1),
      W2, vec(b2), vec(go), vec(bo))


def kernel(x, edge_index, batch,
           W1_0, b1_0, g1_0, be1_0, W2_0, b2_0, eps_0, go_0, bo_0,
           W1_1, b1_1, g1_1, be1_1, W2_1, b2_1, eps_1, go_1, bo_1):
    src = edge_index[0]
    dst = edge_index[1]

    r0 = _relu(x)
    parts0 = _sc_scatter(r0, src, dst)
    h1 = _mlp(x, parts0, eps_0,
              W1_0, b1_0, g1_0, be1_0, W2_0, b2_0, go_0, bo_0,
              final_relu=True)
    parts1 = _sc_scatter(h1, src, dst)
    h2 = _mlp(h1, parts1, eps_1,
              W1_1, b1_1, g1_1, be1_1, W2_1, b2_1, go_1, bo_1,
              final_relu=False)
    return h2


# speedup vs baseline: 12.9463x; 1.1362x over previous
import functools

import jax
import jax.numpy as jnp
from jax import lax
from jax.experimental import pallas as pl
from jax.experimental.pallas import tpu as pltpu
from jax.experimental.pallas import tpu_sc as plsc

N = 10000
D = 128
E = 320000
NC = 2
NS = 16
NW = NC * NS
EPW = E // NW
K = 80
NCHUNK = EPW // K
RPT = 624
RTAIL = N - NS * RPT


def _sc_scatter_body(r_hbm, src_hbm, dst_hbm, out_hbm,
                     q0, q1, q2, q3,
                     rows0, rows1, rows2, rows3, acc_sh,
                     qs0, qs1, qs2, qs3, gs0, gs1, gs2, gs3,
                     ss0, ss1, ss2, ss3):
    c = lax.axis_index("c")
    s = lax.axis_index("s")
    w = s * NC + c
    ebase = w * EPW
    Q = (q0, q1, q2, q3)
    SQ = tuple(q.at[0] for q in Q)
    DQ = tuple(q.at[1] for q in Q)
    QS = (qs0, qs1, qs2, qs3)
    R = (rows0, rows1, rows2, rows3)
    GS = (gs0, gs1, gs2, gs3)
    SS = (ss0, ss1, ss2, ss3)
    LAST = NCHUNK - 1

    def q_start(i, k):
        base = pl.multiple_of(ebase + i * K, 8)
        pltpu.async_copy(src_hbm.at[pl.ds(base, K)], SQ[k], QS[k])
        pltpu.async_copy(dst_hbm.at[pl.ds(base, K)], DQ[k], QS[k])

    def q_wait(k):
        pltpu.make_async_copy(src_hbm.at[pl.ds(0, K)], SQ[k], QS[k]).wait()
        pltpu.make_async_copy(dst_hbm.at[pl.ds(0, K)], DQ[k], QS[k]).wait()

    def g_start(i, k):
        pltpu.async_copy(r_hbm.at[SQ[k]], R[k], GS[k])

    def g_wait(k):
        pltpu.make_async_copy(r_hbm.at[SQ[k]], R[k], GS[k]).wait()

    def s_start(k):
        pltpu.async_copy(R[k], acc_sh.at[DQ[k]], SS[k], add=True)

    def s_wait(k):
        pltpu.make_async_copy(R[k], acc_sh.at[DQ[k]], SS[k]).wait()

    q_start(0, 0)
    q_start(1, 1)
    q_start(2, 2)

    zv = jnp.zeros((16,), jnp.float32)

    def zrow(i, carry):
        for b in range(D // 16):
            rows1[i, pl.ds(b * 16, 16)] = zv
        return carry

    lax.fori_loop(0, K, zrow, 0)
    r0 = pl.multiple_of(s * RPT, 8)
    for t in range(RPT // K):
        pltpu.sync_copy(rows1, acc_sh.at[pl.ds(r0 + t * K, K)])
    pltpu.sync_copy(rows1.at[pl.ds(0, RPT - (RPT // K) * K)],
                    acc_sh.at[pl.ds(r0 + (RPT // K) * K, RPT - (RPT // K) * K)])

    @pl.when(s == 0)
    def _zero_tail():
        pltpu.sync_copy(rows1.at[pl.ds(0, RTAIL)],
                        acc_sh.at[pl.ds(NS * RPT, RTAIL)])

    q_wait(0)
    g_start(0, 0)
    plsc.subcore_barrier()

    q_wait(1); g_start(1, 1); g_wait(0); s_start(0); q_start(3, 3)
    q_wait(2); g_start(2, 2); g_wait(1); s_start(1); s_wait(0); q_start(4, 0)
    q_wait(3); g_start(3, 3); g_wait(2); s_start(2); s_wait(1); q_start(5, 1)

    def quad(j, carry):
        i0 = 3 + 4 * j
        for b in range(4):
            i = i0 + b
            k = (3 + b) % 4
            q_wait((k + 1) % 4)
            g_start(i + 1, (k + 1) % 4)
            g_wait(k)
            s_start(k)
            s_wait((k + 3) % 4)
            q_start(jnp.minimum(i + 3, LAST), (k + 3) % 4)
        return carry

    lax.fori_loop(0, (NCHUNK - 5) // 4, quad, 0)
    q_wait(0); g_start(LAST, 0); g_wait(3); s_start(3); s_wait(2)
    g_wait(0); s_start(0); s_wait(3)
    s_wait(0)
    q_wait(1)
    plsc.subcore_barrier()

    pltpu.sync_copy(acc_sh.at[pl.ds(r0, RPT)], out_hbm.at[c, pl.ds(r0, RPT)])

    @pl.when(s == 0)
    def _write_tail():
        pltpu.sync_copy(acc_sh.at[pl.ds(NS * RPT, RTAIL)],
                        out_hbm.at[c, pl.ds(NS * RPT, RTAIL)])


@functools.cache
def _make_sc_scatter():
    return pl.kernel(
        _sc_scatter_body,
        out_type=jax.ShapeDtypeStruct((NC, N, D), jnp.float32),
        mesh=plsc.VectorSubcoreMesh(core_axis_name="c", subcore_axis_name="s",
                                    num_cores=NC, num_subcores=NS),
        scratch_types=[
            pltpu.VMEM((2, K), jnp.int32),
            pltpu.VMEM((2, K), jnp.int32),
            pltpu.VMEM((2, K), jnp.int32),
            pltpu.VMEM((2, K), jnp.int32),
            pltpu.VMEM((K, D), jnp.float32),
            pltpu.VMEM((K, D), jnp.float32),
            pltpu.VMEM((K, D), jnp.float32),
            pltpu.VMEM((K, D), jnp.float32),
            pltpu.VMEM_SHARED((N, D), jnp.float32),
        ] + [pltpu.SemaphoreType.DMA] * 12,
    )


def _sc_scatter(r, src, dst):
    return _make_sc_scatter()(r, src, dst)


def _relu_body(x_ref, o_ref):
    o_ref[...] = jnp.maximum(x_ref[...], 0.0)


def _relu(x):
    return pl.pallas_call(
        _relu_body,
        out_shape=jax.ShapeDtypeStruct((N, D), jnp.float32),
        grid=(10,),
        in_specs=[pl.BlockSpec((N // 10, D), lambda i: (i, 0))],
        out_specs=pl.BlockSpec((N // 10, D), lambda i: (i, 0)),
    )(x)


def _mlp_body(h_ref, parts_ref, eps_ref, W1_ref, b1_ref, g1_ref,
              be1_ref, W2_ref, b2_ref, go_ref, bo_ref, o_ref, *, final_relu):
    z = (1.0 + eps_ref[0]) * h_ref[...] + (parts_ref[0] + parts_ref[1])
    z1 = jnp.dot(z, W1_ref[...], preferred_element_type=jnp.float32) + b1_ref[...]
    m1 = jnp.mean(z1, axis=0, keepdims=True)
    v1 = jnp.mean((z1 - m1) ** 2, axis=0, keepdims=True)
    z1 = (z1 - m1) * lax.rsqrt(v1 + 1e-5) * g1_ref[...] + be1_ref[...]
    z1 = jnp.maximum(z1, 0.0)
    z2 = jnp.dot(z1, W2_ref[...], preferred_element_type=jnp.float32) + b2_ref[...]
    m2 = jnp.mean(z2, axis=0, keepdims=True)
    v2 = jnp.mean((z2 - m2) ** 2, axis=0, keepdims=True)
    z2 = (z2 - m2) * lax.rsqrt(v2 + 1e-5) * go_ref[...] + bo_ref[...]
    if final_relu:
        z2 = jnp.maximum(z2, 0.0)
    o_ref[...] = z2


def _mlp(h, parts, eps, W1, b1, g1, be1, W2, b2, go, bo, final_relu):
    vec = lambda v: v.reshape(1, D)
    return pl.pallas_call(
        functools.partial(_mlp_body, final_relu=final_relu),
        out_shape=jax.ShapeDtypeStruct((N, D), jnp.float32),
        in_specs=[pl.BlockSpec(memory_space=pltpu.VMEM)] * 2
                 + [pl.BlockSpec(memory_space=pltpu.SMEM)]
                 + [pl.BlockSpec(memory_space=pltpu.VMEM)] * 8,
        out_specs=pl.BlockSpec(memory_space=pltpu.VMEM),
    )(h, parts, eps.reshape(1), W1, vec(b1), vec(g1), vec(be1),
      W2, vec(b2), vec(go), vec(bo))


def kernel(x, edge_index, batch,
           W1_0, b1_0, g1_0, be1_0, W2_0, b2_0, eps_0, go_0, bo_0,
           W1_1, b1_1, g1_1, be1_1, W2_1, b2_1, eps_1, go_1, bo_1):
    src = edge_index[0]
    dst = edge_index[1]

    r0 = _relu(x)
    parts0 = _sc_scatter(r0, src, dst)
    h1 = _mlp(x, parts0, eps_0,
              W1_0, b1_0, g1_0, be1_0, W2_0, b2_0, go_0, bo_0,
              final_relu=True)
    parts1 = _sc_scatter(h1, src, dst)
    h2 = _mlp(h1, parts1, eps_1,
              W1_1, b1_1, g1_1, be1_1, W2_1, b2_1, go_1, bo_1,
              final_relu=False)
    return h2


# speedup vs baseline: 13.5732x; 1.0484x over previous
import functools

import jax
import jax.numpy as jnp
from jax import lax
from jax.experimental import pallas as pl
from jax.experimental.pallas import tpu as pltpu
from jax.experimental.pallas import tpu_sc as plsc

N = 10000
D = 128
E = 320000
NC = 2
NS = 16
NW = NC * NS
EPW = E // NW
K = 80
NCHUNK = EPW // K
RPT = 624
RTAIL = N - NS * RPT


def _sc_scatter_body(r_hbm, src_hbm, dst_hbm, out_hbm,
                     q0, q1, q2, q3,
                     rows0, rows1, rows2, rows3, acc_sh,
                     qs0, qs1, qs2, qs3, gs0, gs1, gs2, gs3,
                     ss0, ss1, ss2, ss3):
    c = lax.axis_index("c")
    s = lax.axis_index("s")
    w = s * NC + c
    ebase = w * EPW
    Q = (q0, q1, q2, q3)
    SQ = tuple(q.at[0] for q in Q)
    DQ = tuple(q.at[1] for q in Q)
    QS = (qs0, qs1, qs2, qs3)
    R = (rows0, rows1, rows2, rows3)
    GS = (gs0, gs1, gs2, gs3)
    SS = (ss0, ss1, ss2, ss3)
    LAST = NCHUNK - 1

    def q_start(i, k):
        base = pl.multiple_of(ebase + i * K, 8)
        pltpu.async_copy(src_hbm.at[pl.ds(base, K)], SQ[k], QS[k])
        pltpu.async_copy(dst_hbm.at[pl.ds(base, K)], DQ[k], QS[k])

    def q_wait(k):
        pltpu.make_async_copy(src_hbm.at[pl.ds(0, K)], SQ[k], QS[k]).wait()
        pltpu.make_async_copy(dst_hbm.at[pl.ds(0, K)], DQ[k], QS[k]).wait()

    def g_start(i, k):
        pltpu.async_copy(r_hbm.at[SQ[k]], R[k], GS[k])

    def g_wait(k):
        pltpu.make_async_copy(r_hbm.at[SQ[k]], R[k], GS[k]).wait()

    def s_start(k):
        pltpu.async_copy(R[k], acc_sh.at[DQ[k]], SS[k], add=True)

    def s_wait(k):
        pltpu.make_async_copy(R[k], acc_sh.at[DQ[k]], SS[k]).wait()

    q_start(0, 0)
    q_start(1, 1)
    q_start(2, 2)

    zv = jnp.zeros((16,), jnp.float32)

    def zrow(i, carry):
        for b in range(D // 16):
            rows1[i, pl.ds(b * 16, 16)] = zv
        return carry

    lax.fori_loop(0, K, zrow, 0)
    r0 = pl.multiple_of(s * RPT, 8)
    for t in range(RPT // K):
        pltpu.sync_copy(rows1, acc_sh.at[pl.ds(r0 + t * K, K)])
    pltpu.sync_copy(rows1.at[pl.ds(0, RPT - (RPT // K) * K)],
                    acc_sh.at[pl.ds(r0 + (RPT // K) * K, RPT - (RPT // K) * K)])

    @pl.when(s == 0)
    def _zero_tail():
        pltpu.sync_copy(rows1.at[pl.ds(0, RTAIL)],
                        acc_sh.at[pl.ds(NS * RPT, RTAIL)])

    q_wait(0)
    g_start(0, 0)
    plsc.subcore_barrier()

    q_wait(1); g_start(1, 1); g_wait(0); s_start(0); q_start(3, 3)
    q_wait(2); g_start(2, 2); g_wait(1); s_start(1); s_wait(0); q_start(4, 0)
    q_wait(3); g_start(3, 3); g_wait(2); s_start(2); s_wait(1); q_start(5, 1)

    def quad(j, carry):
        i0 = 3 + 4 * j
        for b in range(4):
            i = i0 + b
            k = (3 + b) % 4
            q_wait((k + 1) % 4)
            g_start(i + 1, (k + 1) % 4)
            g_wait(k)
            s_start(k)
            s_wait((k + 3) % 4)
            q_start(jnp.minimum(i + 3, LAST), (k + 3) % 4)
        return carry

    lax.fori_loop(0, (NCHUNK - 5) // 4, quad, 0)
    q_wait(0); g_start(LAST, 0); g_wait(3); s_start(3); s_wait(2)
    g_wait(0); s_start(0); s_wait(3)
    s_wait(0)
    q_wait(1)
    plsc.subcore_barrier()

    pltpu.sync_copy(acc_sh.at[pl.ds(r0, RPT)], out_hbm.at[c, pl.ds(r0, RPT)])

    @pl.when(s == 0)
    def _write_tail():
        pltpu.sync_copy(acc_sh.at[pl.ds(NS * RPT, RTAIL)],
                        out_hbm.at[c, pl.ds(NS * RPT, RTAIL)])


@functools.cache
def _make_sc_scatter():
    return pl.kernel(
        _sc_scatter_body,
        out_type=jax.ShapeDtypeStruct((NC, N, D), jnp.float32),
        mesh=plsc.VectorSubcoreMesh(core_axis_name="c", subcore_axis_name="s",
                                    num_cores=NC, num_subcores=NS),
        scratch_types=[
            pltpu.VMEM((2, K), jnp.int32),
            pltpu.VMEM((2, K), jnp.int32),
            pltpu.VMEM((2, K), jnp.int32),
            pltpu.VMEM((2, K), jnp.int32),
            pltpu.VMEM((K, D), jnp.float32),
            pltpu.VMEM((K, D), jnp.float32),
            pltpu.VMEM((K, D), jnp.float32),
            pltpu.VMEM((K, D), jnp.float32),
            pltpu.VMEM_SHARED((N, D), jnp.float32),
        ] + [pltpu.SemaphoreType.DMA] * 12,
    )


def _sc_scatter(r, src, dst):
    return _make_sc_scatter()(r, src, dst)


def _relu_body(x_ref, o_ref):
    o_ref[...] = jnp.maximum(x_ref[...], 0.0)


def _relu(x):
    return pl.pallas_call(
        _relu_body,
        out_shape=jax.ShapeDtypeStruct((N, D), jnp.float32),
        grid=(10,),
        in_specs=[pl.BlockSpec((N // 10, D), lambda i: (i, 0))],
        out_specs=pl.BlockSpec((N // 10, D), lambda i: (i, 0)),
    )(x)


def _split_body(ei_ref, src_ref, dst_ref):
    src_ref[...] = ei_ref[0]
    dst_ref[...] = ei_ref[1]


def _split(edge_index):
    return pl.pallas_call(
        _split_body,
        out_shape=(jax.ShapeDtypeStruct((E,), jnp.int32),
                   jax.ShapeDtypeStruct((E,), jnp.int32)),
    )(edge_index)


def _mlp_body(h_ref, parts_ref, eps_ref, W1_ref, b1_ref, g1_ref,
              be1_ref, W2_ref, b2_ref, go_ref, bo_ref, o_ref, *, final_relu):
    z = (1.0 + eps_ref[0]) * h_ref[...] + (parts_ref[0] + parts_ref[1])
    z1 = jnp.dot(z, W1_ref[...], preferred_element_type=jnp.float32) + b1_ref[...]
    m1 = jnp.mean(z1, axis=0, keepdims=True)
    v1 = jnp.mean((z1 - m1) ** 2, axis=0, keepdims=True)
    z1 = (z1 - m1) * lax.rsqrt(v1 + 1e-5) * g1_ref[...] + be1_ref[...]
    z1 = jnp.maximum(z1, 0.0)
    z2 = jnp.dot(z1, W2_ref[...], preferred_element_type=jnp.float32) + b2_ref[...]
    m2 = jnp.mean(z2, axis=0, keepdims=True)
    v2 = jnp.mean((z2 - m2) ** 2, axis=0, keepdims=True)
    z2 = (z2 - m2) * lax.rsqrt(v2 + 1e-5) * go_ref[...] + bo_ref[...]
    if final_relu:
        z2 = jnp.maximum(z2, 0.0)
    o_ref[...] = z2


def _mlp(h, parts, eps, W1, b1, g1, be1, W2, b2, go, bo, final_relu):
    vec = lambda v: v.reshape(1, D)
    return pl.pallas_call(
        functools.partial(_mlp_body, final_relu=final_relu),
        out_shape=jax.ShapeDtypeStruct((N, D), jnp.float32),
        in_specs=[pl.BlockSpec(memory_space=pltpu.VMEM)] * 2
                 + [pl.BlockSpec(memory_space=pltpu.SMEM)]
                 + [pl.BlockSpec(memory_space=pltpu.VMEM)] * 8,
        out_specs=pl.BlockSpec(memory_space=pltpu.VMEM),
    )(h, parts, eps.reshape(1), W1, vec(b1), vec(g1), vec(be1),
      W2, vec(b2), vec(go), vec(bo))


def kernel(x, edge_index, batch,
           W1_0, b1_0, g1_0, be1_0, W2_0, b2_0, eps_0, go_0, bo_0,
           W1_1, b1_1, g1_1, be1_1, W2_1, b2_1, eps_1, go_1, bo_1):
    src, dst = _split(edge_index)
    r0 = _relu(x)
    parts0 = _sc_scatter(r0, src, dst)
    h1 = _mlp(x, parts0, eps_0,
              W1_0, b1_0, g1_0, be1_0, W2_0, b2_0, go_0, bo_0,
              final_relu=True)
    parts1 = _sc_scatter(h1, src, dst)
    h2 = _mlp(h1, parts1, eps_1,
              W1_1, b1_1, g1_1, be1_1, W2_1, b2_1, go_1, bo_1,
              final_relu=False)
    return h2


# speedup vs baseline: 13.6507x; 1.0057x over previous
import functools

import jax
import jax.numpy as jnp
from jax import lax
from jax.experimental import pallas as pl
from jax.experimental.pallas import tpu as pltpu
from jax.experimental.pallas import tpu_sc as plsc

N = 10000
D = 128
E = 320000
NC = 2
NS = 16
NW = NC * NS
EPW = E // NW
K = 80
NCHUNK = EPW // K
RPT = 624
RTAIL = N - NS * RPT


def _sc_scatter_body(r_hbm, src_hbm, dst_hbm, out_hbm,
                     q0, q1, q2, q3,
                     rows0, rows1, rows2, rows3, acc_sh,
                     qs0, qs1, qs2, qs3, gs0, gs1, gs2, gs3,
                     ss0, ss1, ss2, ss3):
    c = lax.axis_index("c")
    s = lax.axis_index("s")
    w = s * NC + c
    ebase = w * EPW
    Q = (q0, q1, q2, q3)
    SQ = tuple(q.at[0] for q in Q)
    DQ = tuple(q.at[1] for q in Q)
    QS = (qs0, qs1, qs2, qs3)
    R = (rows0, rows1, rows2, rows3)
    GS = (gs0, gs1, gs2, gs3)
    SS = (ss0, ss1, ss2, ss3)
    LAST = NCHUNK - 1

    def q_start(i, k):
        base = pl.multiple_of(ebase + i * K, 8)
        pltpu.async_copy(src_hbm.at[pl.ds(base, K)], SQ[k], QS[k])
        pltpu.async_copy(dst_hbm.at[pl.ds(base, K)], DQ[k], QS[k])

    def q_wait(k):
        pltpu.make_async_copy(src_hbm.at[pl.ds(0, K)], SQ[k], QS[k]).wait()
        pltpu.make_async_copy(dst_hbm.at[pl.ds(0, K)], DQ[k], QS[k]).wait()

    def g_start(i, k):
        pltpu.async_copy(r_hbm.at[SQ[k]], R[k], GS[k])

    def g_wait(k):
        pltpu.make_async_copy(r_hbm.at[SQ[k]], R[k], GS[k]).wait()

    def s_start(k):
        pltpu.async_copy(R[k], acc_sh.at[DQ[k]], SS[k], add=True)

    def s_wait(k):
        pltpu.make_async_copy(R[k], acc_sh.at[DQ[k]], SS[k]).wait()

    q_start(0, 0)
    q_start(1, 1)
    q_start(2, 2)

    zv = jnp.zeros((16,), jnp.float32)

    def zrow(i, carry):
        for b in range(D // 16):
            rows1[i, pl.ds(b * 16, 16)] = zv
        return carry

    lax.fori_loop(0, K, zrow, 0)
    r0 = pl.multiple_of(s * RPT, 8)
    ZREM = RPT - (RPT // K) * K
    for t in range(RPT // K):
        pltpu.async_copy(rows1, acc_sh.at[pl.ds(r0 + t * K, K)], ss0)
    pltpu.async_copy(rows1.at[pl.ds(0, ZREM)],
                     acc_sh.at[pl.ds(r0 + (RPT // K) * K, ZREM)], ss0)

    @pl.when(s == 0)
    def _zero_tail():
        pltpu.sync_copy(rows1.at[pl.ds(0, RTAIL)],
                        acc_sh.at[pl.ds(NS * RPT, RTAIL)])

    q_wait(0)
    g_start(0, 0)
    for t in range(RPT // K):
        pltpu.make_async_copy(rows1, acc_sh.at[pl.ds(r0 + t * K, K)], ss0).wait()
    pltpu.make_async_copy(rows1.at[pl.ds(0, ZREM)],
                          acc_sh.at[pl.ds(r0 + (RPT // K) * K, ZREM)], ss0).wait()
    plsc.subcore_barrier()

    q_wait(1); g_start(1, 1); g_wait(0); s_start(0); q_start(3, 3)
    q_wait(2); g_start(2, 2); g_wait(1); s_start(1); s_wait(0); q_start(4, 0)
    q_wait(3); g_start(3, 3); g_wait(2); s_start(2); s_wait(1); q_start(5, 1)

    def quad(j, carry):
        i0 = 3 + 4 * j
        for b in range(4):
            i = i0 + b
            k = (3 + b) % 4
            q_wait((k + 1) % 4)
            g_start(i + 1, (k + 1) % 4)
            g_wait(k)
            s_start(k)
            s_wait((k + 3) % 4)
            q_start(jnp.minimum(i + 3, LAST), (k + 3) % 4)
        return carry

    lax.fori_loop(0, (NCHUNK - 5) // 4, quad, 0)
    q_wait(0); g_start(LAST, 0); g_wait(3); s_start(3); s_wait(2)
    g_wait(0); s_start(0); s_wait(3)
    s_wait(0)
    q_wait(1)
    plsc.subcore_barrier()

    pltpu.sync_copy(acc_sh.at[pl.ds(r0, RPT)], out_hbm.at[c, pl.ds(r0, RPT)])

    @pl.when(s == 0)
    def _write_tail():
        pltpu.sync_copy(acc_sh.at[pl.ds(NS * RPT, RTAIL)],
                        out_hbm.at[c, pl.ds(NS * RPT, RTAIL)])


@functools.cache
def _make_sc_scatter():
    return pl.kernel(
        _sc_scatter_body,
        out_type=jax.ShapeDtypeStruct((NC, N, D), jnp.float32),
        mesh=plsc.VectorSubcoreMesh(core_axis_name="c", subcore_axis_name="s",
                                    num_cores=NC, num_subcores=NS),
        scratch_types=[
            pltpu.VMEM((2, K), jnp.int32),
            pltpu.VMEM((2, K), jnp.int32),
            pltpu.VMEM((2, K), jnp.int32),
            pltpu.VMEM((2, K), jnp.int32),
            pltpu.VMEM((K, D), jnp.float32),
            pltpu.VMEM((K, D), jnp.float32),
            pltpu.VMEM((K, D), jnp.float32),
            pltpu.VMEM((K, D), jnp.float32),
            pltpu.VMEM_SHARED((N, D), jnp.float32),
        ] + [pltpu.SemaphoreType.DMA] * 12,
    )


def _sc_scatter(r, src, dst):
    return _make_sc_scatter()(r, src, dst)


def _relu_body(x_ref, o_ref):
    o_ref[...] = jnp.maximum(x_ref[...], 0.0)


def _relu(x):
    return pl.pallas_call(
        _relu_body,
        out_shape=jax.ShapeDtypeStruct((N, D), jnp.float32),
        grid=(10,),
        in_specs=[pl.BlockSpec((N // 10, D), lambda i: (i, 0))],
        out_specs=pl.BlockSpec((N // 10, D), lambda i: (i, 0)),
    )(x)


def _split_body(ei_ref, src_ref, dst_ref):
    src_ref[...] = ei_ref[0]
    dst_ref[...] = ei_ref[1]


def _split(edge_index):
    return pl.pallas_call(
        _split_body,
        out_shape=(jax.ShapeDtypeStruct((E,), jnp.int32),
                   jax.ShapeDtypeStruct((E,), jnp.int32)),
    )(edge_index)


def _mlp_body(h_ref, parts_ref, eps_ref, W1_ref, b1_ref, g1_ref,
              be1_ref, W2_ref, b2_ref, go_ref, bo_ref, o_ref, *, final_relu):
    z = (1.0 + eps_ref[0]) * h_ref[...] + (parts_ref[0] + parts_ref[1])
    z1 = jnp.dot(z, W1_ref[...], preferred_element_type=jnp.float32) + b1_ref[...]
    m1 = jnp.mean(z1, axis=0, keepdims=True)
    v1 = jnp.mean((z1 - m1) ** 2, axis=0, keepdims=True)
    z1 = (z1 - m1) * lax.rsqrt(v1 + 1e-5) * g1_ref[...] + be1_ref[...]
    z1 = jnp.maximum(z1, 0.0)
    z2 = jnp.dot(z1, W2_ref[...], preferred_element_type=jnp.float32) + b2_ref[...]
    m2 = jnp.mean(z2, axis=0, keepdims=True)
    v2 = jnp.mean((z2 - m2) ** 2, axis=0, keepdims=True)
    z2 = (z2 - m2) * lax.rsqrt(v2 + 1e-5) * go_ref[...] + bo_ref[...]
    if final_relu:
        z2 = jnp.maximum(z2, 0.0)
    o_ref[...] = z2


def _mlp(h, parts, eps, W1, b1, g1, be1, W2, b2, go, bo, final_relu):
    vec = lambda v: v.reshape(1, D)
    return pl.pallas_call(
        functools.partial(_mlp_body, final_relu=final_relu),
        out_shape=jax.ShapeDtypeStruct((N, D), jnp.float32),
        in_specs=[pl.BlockSpec(memory_space=pltpu.VMEM)] * 2
                 + [pl.BlockSpec(memory_space=pltpu.SMEM)]
                 + [pl.BlockSpec(memory_space=pltpu.VMEM)] * 8,
        out_specs=pl.BlockSpec(memory_space=pltpu.VMEM),
    )(h, parts, eps.reshape(1), W1, vec(b1), vec(g1), vec(be1),
      W2, vec(b2), vec(go), vec(bo))


def kernel(x, edge_index, batch,
           W1_0, b1_0, g1_0, be1_0, W2_0, b2_0, eps_0, go_0, bo_0,
           W1_1, b1_1, g1_1, be1_1, W2_1, b2_1, eps_1, go_1, bo_1):
    src, dst = _split(edge_index)
    r0 = _relu(x)
    parts0 = _sc_scatter(r0, src, dst)
    h1 = _mlp(x, parts0, eps_0,
              W1_0, b1_0, g1_0, be1_0, W2_0, b2_0, go_0, bo_0,
              final_relu=True)
    parts1 = _sc_scatter(h1, src, dst)
    h2 = _mlp(h1, parts1, eps_1,
              W1_1, b1_1, g1_1, be1_1, W2_1, b2_1, go_1, bo_1,
              final_relu=False)
    return h2
